# baseline (speedup 1.0000x reference)
import functools

import jax
import jax.numpy as jnp
from jax import lax
from jax.experimental import pallas as pl
from jax.experimental.pallas import tpu as pltpu
from jax.experimental.pallas import tpu_sc as plsc

N_PREFIX = 10000
N_AS = 10000
N_EDGES = 320000
EMBED = 64

NC = 2
NS = 16
NW = NC * NS
CH = 128
E_PAD = N_EDGES
NCHUNKS = E_PAD // CH

WIN = 1280
NWIN = 8
ACCR = 1296
TRASH = WIN
NBUF = 8
MAXP = 4

N_OUT = NWIN * WIN


def _fmap_body(x_as_ref, w_as_ref, b_as_ref, w_att_ref, a2_ref, u1d_ref,
               as_out_ref, g_ref, b_ref):
    x = x_as_ref[...]
    h = x @ w_as_ref[...] + b_as_ref[...]
    h = jnp.maximum(h, 0.0)
    nrm = jnp.sqrt(jnp.sum(h * h, axis=1, keepdims=True))
    as_out_ref[...] = h / jnp.maximum(nrm, 1e-12)
    w2 = w_att_ref[...] @ a2_ref[...]
    s = h @ w2
    es = jnp.exp(s - jnp.max(s))
    g_ref[...] = es * h
    u2 = u1d_ref[...].reshape(NCHUNKS, CH)
    cmin = u2[:, 0:1]
    cmax = u2[:, CH - 1:CH]
    lane = lax.broadcasted_iota(jnp.int32, (1, 128), 1)
    row = jnp.zeros((1, 128), jnp.int32)
    for w in range(NWIN):
        c_lo = jnp.sum((cmax < w * WIN).astype(jnp.int32))
        c_hi = jnp.sum((cmin < (w + 1) * WIN).astype(jnp.int32))
        row = jnp.where(lane == w, c_lo, row)
        row = jnp.where(lane == 8 + w, c_hi, row)
    b_ref[...] = row


def _fmap_call(x_as, w_as, b_as, w_att, a2, u1d):
    return pl.pallas_call(
        _fmap_body,
        out_shape=(
            jax.ShapeDtypeStruct((N_AS, EMBED), jnp.float32),
            jax.ShapeDtypeStruct((N_AS, EMBED), jnp.float32),
            jax.ShapeDtypeStruct((1, 128), jnp.int32),
        ),
    )(x_as, w_as, b_as, w_att, a2, u1d)


_MESH = plsc.VectorSubcoreMesh(core_axis_name="c", subcore_axis_name="s")


@functools.partial(
    pl.kernel,
    out_type=jax.ShapeDtypeStruct((NWIN, WIN, EMBED), jnp.float32),
    mesh=_MESH,
    compiler_params=pltpu.CompilerParams(use_tc_tiling_on_sc=False, needs_layout_passes=False),
    scratch_types=[
        pltpu.VMEM((NBUF, CH), jnp.int32),
        pltpu.VMEM((NBUF, CH), jnp.int32),
        pltpu.VMEM((NBUF, CH), jnp.int32),
        pltpu.VMEM((NBUF, CH, EMBED), jnp.float32),
        pltpu.VMEM((ACCR // NS, EMBED), jnp.float32),
        pltpu.VMEM((16,), jnp.int32),
        pltpu.VMEM_SHARED((ACCR, EMBED), jnp.float32),
        pltpu.VMEM_SHARED((N_AS, EMBED), jnp.float32),
        pltpu.SemaphoreType.DMA,
        pltpu.SemaphoreType.DMA,
        pltpu.SemaphoreType.DMA,
    ],
)
def _edge_kernel(u1, v1, g, b2d, out,
                 vbuf, ubuf, tbuf, rows, zbuf, bsm, acc, gsp,
                 isem, gsem, ssem):
    c = lax.axis_index("c")
    s = lax.axis_index("s")
    pltpu.sync_copy(b2d.at[0, pl.ds(0, 16)], bsm)
    gpt = N_AS // NS
    pltpu.sync_copy(g.at[pl.ds(s * gpt, gpt)], gsp.at[pl.ds(s * gpt, gpt)])
    for r in range(ACCR // NS):
        for q in range(EMBED // 16):
            zbuf[r, pl.ds(q * 16, 16)] = jnp.zeros((16,), jnp.float32)

    def start_idx(k, r):
        pltpu.async_copy(u1.at[pl.ds(k * CH, CH)], ubuf.at[r], isem)
        pltpu.async_copy(v1.at[pl.ds(k * CH, CH)], vbuf.at[r], isem)

    def wait_idx(k, r):
        pltpu.make_async_copy(u1.at[pl.ds(k * CH, CH)], ubuf.at[r], isem).wait()
        pltpu.make_async_copy(v1.at[pl.ds(k * CH, CH)], vbuf.at[r], isem).wait()

    def compute_t(r, u_lo):
        for q in range(CH // 16):
            u_vec = ubuf[r, pl.ds(q * 16, 16)]
            d = u_vec - u_lo
            in_win = plsc.bitcast(d, jnp.uint32) < jnp.uint32(WIN)
            tbuf[r, pl.ds(q * 16, 16)] = jnp.where(in_win, d, TRASH)

    for p in range(MAXP):
        w = 2 * p + c

        @pl.when(w < NWIN)
        def _pass():
            u_lo = w * WIN
            bvec = bsm[...]
            lane16 = lax.iota(jnp.int32, 16)
            c_lo = jnp.max(jnp.where(lane16 == w, bvec, 0))
            c_hi = jnp.max(jnp.where(lane16 == 8 + w, bvec, 0))
            nch = c_hi - c_lo
            n_j = lax.shift_right_arithmetic(nch - s + 15, 4)
            base = c_lo + s

            pltpu.sync_copy(zbuf.at[pl.ds(0, ACCR // NS)],
                            acc.at[pl.ds(s * (ACCR // NS), ACCR // NS)])
            plsc.subcore_barrier()

            for jj in range(5):
                @pl.when(jj < n_j)
                def _():
                    start_idx(base + 16 * jj, jj)
            for jj in range(3):
                @pl.when(jj < n_j)
                def _():
                    wait_idx(base + 16 * jj, jj)
                    compute_t(jj, u_lo)
                    pltpu.async_copy(gsp.at[vbuf.at[jj]], rows.at[jj], gsem)

            def body(j, carry):
                @pl.when(j + 5 < n_j)
                def _():
                    start_idx(base + 16 * (j + 5), lax.rem(j + 5, NBUF))

                @pl.when(j + 3 < n_j)
                def _():
                    r2 = lax.rem(j + 3, NBUF)
                    wait_idx(base + 16 * (j + 3), r2)
                    compute_t(r2, u_lo)
                    pltpu.async_copy(gsp.at[vbuf.at[r2]], rows.at[r2], gsem)

                r0 = lax.rem(j, NBUF)
                pltpu.make_async_copy(gsp.at[vbuf.at[r0]], rows.at[r0],
                                      gsem).wait()
                pltpu.async_copy(rows.at[r0], acc.at[tbuf.at[r0]], ssem,
                                 add=True)

                @pl.when(j >= 3)
                def _():
                    rm = lax.rem(j - 3, NBUF)
                    pltpu.make_async_copy(rows.at[rm], acc.at[tbuf.at[rm]],
                                          ssem).wait()
                return carry

            lax.fori_loop(0, n_j, body, 0)
            for jj in range(3):
                @pl.when(jj < n_j)
                def _():
                    rt = lax.rem(n_j - 1 - jj, NBUF)
                    pltpu.make_async_copy(rows.at[rt], acc.at[tbuf.at[rt]],
                                          ssem).wait()
            plsc.subcore_barrier()

            pltpu.sync_copy(acc.at[pl.ds(s * (WIN // NS), WIN // NS)],
                            out.at[w, pl.ds(s * (WIN // NS), WIN // NS)])
            plsc.subcore_barrier()


def _norm_body(acc_ref, out_ref):
    p = acc_ref[...].reshape(N_OUT, EMBED)
    nrm = jnp.sqrt(jnp.sum(p * p, axis=1, keepdims=True))
    out_ref[...] = (p / jnp.maximum(nrm, 1e-12))[:N_PREFIX]


def _norm_call(acc9):
    return pl.pallas_call(
        _norm_body,
        out_shape=jax.ShapeDtypeStruct((N_PREFIX, EMBED), jnp.float32),
    )(acc9)


def kernel(x_prefix, x_as, W_prefix, b_prefix, W_as, b_as,
           W_att, b_att, a_att, b_a, edge_u, edge_v):
    del x_prefix, W_prefix, b_prefix, b_att, b_a
    a2 = a_att[EMBED:, :]
    as_out, g_as, b_row = _fmap_call(
        x_as, W_as, b_as.reshape(1, EMBED), W_att, a2, edge_u)

    acc9 = _edge_kernel(edge_u, edge_v, g_as, b_row)
    prefix_out = _norm_call(acc9)
    return prefix_out, as_out


# speedup vs baseline: 1.0173x; 1.0173x over previous
import functools

import jax
import jax.numpy as jnp
from jax import lax
from jax.experimental import pallas as pl
from jax.experimental.pallas import tpu as pltpu
from jax.experimental.pallas import tpu_sc as plsc

N_PREFIX = 10000
N_AS = 10000
N_EDGES = 320000
EMBED = 64

NC = 2
NS = 16
NW = NC * NS
CH = 128
E_PAD = N_EDGES
NCHUNKS = E_PAD // CH

WIN = 1280
NWIN = 8
ACCR = 1296
TRASH = WIN
NBUF = 8
MAXP = 4

N_OUT = NWIN * WIN
GPAD = 10112


def _fmap_body(x_as_ref, w_as_ref, b_as_ref, w_att_ref, a2_ref, u2d_ref,
               as_out_ref, g_ref, b_ref):
    x = x_as_ref[...]
    h = x @ w_as_ref[...] + b_as_ref[...]
    h = jnp.maximum(h, 0.0)
    nrm = jnp.sqrt(jnp.sum(h * h, axis=1, keepdims=True))
    as_out_ref[...] = h / jnp.maximum(nrm, 1e-12)
    w2 = w_att_ref[...] @ a2_ref[...]
    s = h @ w2
    es = jnp.exp(s - jnp.max(s))
    g_ref[...] = es * h
    u2 = u2d_ref[...]
    lane = lax.broadcasted_iota(jnp.int32, (1, 128), 1)
    row = jnp.where(lane == 8 + (NWIN - 1), NCHUNKS, 0)
    for w in range(1, NWIN):
        cnt = jnp.sum((u2 < w * WIN).astype(jnp.int32))
        row = jnp.where(lane == w, lax.shift_right_logical(cnt, 7), row)
        row = jnp.where(lane == 8 + (w - 1),
                        lax.shift_right_logical(cnt + 127, 7), row)
    b_ref[...] = row


def _fmap_call(x_as, w_as, b_as, w_att, a2, u2d):
    return pl.pallas_call(
        _fmap_body,
        out_shape=(
            jax.ShapeDtypeStruct((N_AS, EMBED), jnp.float32),
            jax.ShapeDtypeStruct((N_AS, EMBED), jnp.float32),
            jax.ShapeDtypeStruct((1, 128), jnp.int32),
        ),
    )(x_as, w_as, b_as, w_att, a2, u2d)


_MESH = plsc.VectorSubcoreMesh(core_axis_name="c", subcore_axis_name="s")


@functools.partial(
    pl.kernel,
    out_type=jax.ShapeDtypeStruct((NWIN, WIN, EMBED), jnp.float32),
    mesh=_MESH,
    compiler_params=pltpu.CompilerParams(use_tc_tiling_on_sc=False, needs_layout_passes=False),
    scratch_types=[
        pltpu.VMEM((NBUF, CH), jnp.int32),
        pltpu.VMEM((NBUF, CH), jnp.int32),
        pltpu.VMEM((NBUF, CH), jnp.int32),
        pltpu.VMEM((NBUF, CH, EMBED), jnp.float32),
        pltpu.VMEM((ACCR // NS, EMBED), jnp.float32),
        pltpu.VMEM((16,), jnp.int32),
        pltpu.VMEM_SHARED((ACCR, EMBED), jnp.float32),
        pltpu.VMEM_SHARED((N_AS, EMBED), jnp.float32),
        pltpu.SemaphoreType.DMA,
        pltpu.SemaphoreType.DMA,
        pltpu.SemaphoreType.DMA,
    ],
)
def _edge_kernel(u1, v1, g, b2d, out,
                 vbuf, ubuf, tbuf, rows, zbuf, bsm, acc, gsp,
                 isem, gsem, ssem):
    c = lax.axis_index("c")
    s = lax.axis_index("s")
    pltpu.sync_copy(b2d.at[0, pl.ds(0, 16)], bsm)
    gpt = N_AS // NS
    pltpu.sync_copy(g.at[pl.ds(s * gpt, gpt)], gsp.at[pl.ds(s * gpt, gpt)])
    for r in range(ACCR // NS):
        for q in range(EMBED // 16):
            zbuf[r, pl.ds(q * 16, 16)] = jnp.zeros((16,), jnp.float32)

    def start_idx(k, r):
        pltpu.async_copy(u1.at[pl.ds(k * CH, CH)], ubuf.at[r], isem)
        pltpu.async_copy(v1.at[pl.ds(k * CH, CH)], vbuf.at[r], isem)

    def wait_idx(k, r):
        pltpu.make_async_copy(u1.at[pl.ds(k * CH, CH)], ubuf.at[r], isem).wait()
        pltpu.make_async_copy(v1.at[pl.ds(k * CH, CH)], vbuf.at[r], isem).wait()

    def compute_t(r, u_lo):
        for q in range(CH // 16):
            u_vec = ubuf[r, pl.ds(q * 16, 16)]
            d = u_vec - u_lo
            in_win = plsc.bitcast(d, jnp.uint32) < jnp.uint32(WIN)
            tbuf[r, pl.ds(q * 16, 16)] = jnp.where(in_win, d, TRASH)

    for p in range(MAXP):
        w = 2 * p + c

        @pl.when(w < NWIN)
        def _pass():
            u_lo = w * WIN
            bvec = bsm[...]
            lane16 = lax.iota(jnp.int32, 16)
            c_lo = jnp.max(jnp.where(lane16 == w, bvec, 0))
            c_hi = jnp.max(jnp.where(lane16 == 8 + w, bvec, 0))
            nch = c_hi - c_lo
            n_j = lax.shift_right_arithmetic(nch - s + 15, 4)
            base = c_lo + s

            pltpu.sync_copy(zbuf.at[pl.ds(0, ACCR // NS)],
                            acc.at[pl.ds(s * (ACCR // NS), ACCR // NS)])
            plsc.subcore_barrier()

            for jj in range(5):
                @pl.when(jj < n_j)
                def _():
                    start_idx(base + 16 * jj, jj)
            for jj in range(3):
                @pl.when(jj < n_j)
                def _():
                    wait_idx(base + 16 * jj, jj)
                    compute_t(jj, u_lo)
                    pltpu.async_copy(gsp.at[vbuf.at[jj]], rows.at[jj], gsem)

            def body(j, carry):
                @pl.when(j + 5 < n_j)
                def _():
                    start_idx(base + 16 * (j + 5), lax.rem(j + 5, NBUF))

                @pl.when(j + 3 < n_j)
                def _():
                    r2 = lax.rem(j + 3, NBUF)
                    wait_idx(base + 16 * (j + 3), r2)
                    compute_t(r2, u_lo)
                    pltpu.async_copy(gsp.at[vbuf.at[r2]], rows.at[r2], gsem)

                r0 = lax.rem(j, NBUF)
                pltpu.make_async_copy(gsp.at[vbuf.at[r0]], rows.at[r0],
                                      gsem).wait()
                pltpu.async_copy(rows.at[r0], acc.at[tbuf.at[r0]], ssem,
                                 add=True)

                @pl.when(j >= 3)
                def _():
                    rm = lax.rem(j - 3, NBUF)
                    pltpu.make_async_copy(rows.at[rm], acc.at[tbuf.at[rm]],
                                          ssem).wait()
                return carry

            lax.fori_loop(0, n_j, body, 0)
            for jj in range(3):
                @pl.when(jj < n_j)
                def _():
                    rt = lax.rem(n_j - 1 - jj, NBUF)
                    pltpu.make_async_copy(rows.at[rt], acc.at[tbuf.at[rt]],
                                          ssem).wait()
            plsc.subcore_barrier()

            pltpu.sync_copy(acc.at[pl.ds(s * (WIN // NS), WIN // NS)],
                            out.at[w, pl.ds(s * (WIN // NS), WIN // NS)])
            plsc.subcore_barrier()


def _norm_body(acc_ref, out_ref):
    p = acc_ref[...].reshape(N_OUT, EMBED)
    nrm = jnp.sqrt(jnp.sum(p * p, axis=1, keepdims=True))
    out_ref[...] = (p / jnp.maximum(nrm, 1e-12))[:N_PREFIX]


def _norm_call(acc9):
    return pl.pallas_call(
        _norm_body,
        out_shape=jax.ShapeDtypeStruct((N_PREFIX, EMBED), jnp.float32),
    )(acc9)


def kernel(x_prefix, x_as, W_prefix, b_prefix, W_as, b_as,
           W_att, b_att, a_att, b_a, edge_u, edge_v):
    del x_prefix, W_prefix, b_prefix, b_att, b_a
    a2 = a_att[EMBED:, :]
    as_out, g_as, b_row = _fmap_call(
        x_as, W_as, b_as.reshape(1, EMBED), W_att, a2, edge_u.reshape(-1, CH))

    acc9 = _edge_kernel(edge_u, edge_v, g_as, b_row)
    prefix_out = _norm_call(acc9)
    return prefix_out, as_out


# speedup vs baseline: 1.0624x; 1.0443x over previous
import functools

import jax
import jax.numpy as jnp
from jax import lax
from jax.experimental import pallas as pl
from jax.experimental.pallas import tpu as pltpu
from jax.experimental.pallas import tpu_sc as plsc

N_PREFIX = 10000
N_AS = 10000
N_EDGES = 320000
EMBED = 64

NC = 2
NS = 16
NW = NC * NS
CH = 128
E_PAD = N_EDGES
NCHUNKS = E_PAD // CH

WIN = 2560
NWIN = 4
ACCR = 2576
TRASH = WIN
NBUF = 8
MAXP = 2

N_OUT = NWIN * WIN
GPAD = 10112


def _fmap_body(x_as_ref, w_as_ref, b_as_ref, w_att_ref, a2_ref, u2d_ref,
               as_out_ref, g_ref, b_ref):
    x = x_as_ref[...]
    h = x @ w_as_ref[...] + b_as_ref[...]
    h = jnp.maximum(h, 0.0)
    nrm = jnp.sqrt(jnp.sum(h * h, axis=1, keepdims=True))
    as_out_ref[...] = h / jnp.maximum(nrm, 1e-12)
    w2 = w_att_ref[...] @ a2_ref[...]
    s = h @ w2
    es = jnp.exp(s - jnp.max(s))
    g_ref[...] = es * h
    u2 = u2d_ref[...]
    lane = lax.broadcasted_iota(jnp.int32, (1, 128), 1)
    row = jnp.where(lane == 8 + (NWIN - 1), NCHUNKS, 0)
    for w in range(1, NWIN):
        cnt = jnp.sum((u2 < w * WIN).astype(jnp.int32))
        row = jnp.where(lane == w, lax.shift_right_logical(cnt, 7), row)
        row = jnp.where(lane == 8 + (w - 1),
                        lax.shift_right_logical(cnt + 127, 7), row)
    b_ref[...] = row


def _fmap_call(x_as, w_as, b_as, w_att, a2, u2d):
    return pl.pallas_call(
        _fmap_body,
        out_shape=(
            jax.ShapeDtypeStruct((N_AS, EMBED), jnp.float32),
            jax.ShapeDtypeStruct((N_AS, EMBED), jnp.float32),
            jax.ShapeDtypeStruct((1, 128), jnp.int32),
        ),
    )(x_as, w_as, b_as, w_att, a2, u2d)


_MESH = plsc.VectorSubcoreMesh(core_axis_name="c", subcore_axis_name="s")


@functools.partial(
    pl.kernel,
    out_type=jax.ShapeDtypeStruct((NWIN, WIN, EMBED), jnp.float32),
    mesh=_MESH,
    compiler_params=pltpu.CompilerParams(use_tc_tiling_on_sc=False, needs_layout_passes=False),
    scratch_types=[
        pltpu.VMEM((NBUF, CH), jnp.int32),
        pltpu.VMEM((NBUF, CH), jnp.int32),
        pltpu.VMEM((NBUF, CH), jnp.int32),
        pltpu.VMEM((NBUF, CH, EMBED), jnp.float32),
        pltpu.VMEM((ACCR // NS, EMBED), jnp.float32),
        pltpu.VMEM((16,), jnp.int32),
        pltpu.VMEM_SHARED((ACCR, EMBED), jnp.float32),
        pltpu.VMEM_SHARED((N_AS, EMBED), jnp.float32),
        pltpu.SemaphoreType.DMA,
        pltpu.SemaphoreType.DMA,
        pltpu.SemaphoreType.DMA,
    ],
)
def _edge_kernel(u1, v1, g, b2d, out,
                 vbuf, ubuf, tbuf, rows, zbuf, bsm, acc, gsp,
                 isem, gsem, ssem):
    c = lax.axis_index("c")
    s = lax.axis_index("s")
    pltpu.sync_copy(b2d.at[0, pl.ds(0, 16)], bsm)
    gpt = N_AS // NS
    pltpu.sync_copy(g.at[pl.ds(s * gpt, gpt)], gsp.at[pl.ds(s * gpt, gpt)])
    for r in range(ACCR // NS):
        for q in range(EMBED // 16):
            zbuf[r, pl.ds(q * 16, 16)] = jnp.zeros((16,), jnp.float32)

    def start_idx(k, r):
        pltpu.async_copy(u1.at[pl.ds(k * CH, CH)], ubuf.at[r], isem)
        pltpu.async_copy(v1.at[pl.ds(k * CH, CH)], vbuf.at[r], isem)

    def wait_idx(k, r):
        pltpu.make_async_copy(u1.at[pl.ds(k * CH, CH)], ubuf.at[r], isem).wait()
        pltpu.make_async_copy(v1.at[pl.ds(k * CH, CH)], vbuf.at[r], isem).wait()

    def compute_t(r, u_lo):
        for q in range(CH // 16):
            u_vec = ubuf[r, pl.ds(q * 16, 16)]
            d = u_vec - u_lo
            in_win = plsc.bitcast(d, jnp.uint32) < jnp.uint32(WIN)
            tbuf[r, pl.ds(q * 16, 16)] = jnp.where(in_win, d, TRASH)

    for p in range(MAXP):
        w = 2 * p + c

        @pl.when(w < NWIN)
        def _pass():
            u_lo = w * WIN
            bvec = bsm[...]
            lane16 = lax.iota(jnp.int32, 16)
            c_lo = jnp.max(jnp.where(lane16 == w, bvec, 0))
            c_hi = jnp.max(jnp.where(lane16 == 8 + w, bvec, 0))
            nch = c_hi - c_lo
            n_j = lax.shift_right_arithmetic(nch - s + 15, 4)
            base = c_lo + s

            pltpu.sync_copy(zbuf.at[pl.ds(0, ACCR // NS)],
                            acc.at[pl.ds(s * (ACCR // NS), ACCR // NS)])
            plsc.subcore_barrier()

            for jj in range(5):
                @pl.when(jj < n_j)
                def _():
                    start_idx(base + 16 * jj, jj)
            for jj in range(3):
                @pl.when(jj < n_j)
                def _():
                    wait_idx(base + 16 * jj, jj)
                    compute_t(jj, u_lo)
                    pltpu.async_copy(gsp.at[vbuf.at[jj]], rows.at[jj], gsem)

            def body(j, carry):
                @pl.when(j + 5 < n_j)
                def _():
                    start_idx(base + 16 * (j + 5), lax.rem(j + 5, NBUF))

                @pl.when(j + 3 < n_j)
                def _():
                    r2 = lax.rem(j + 3, NBUF)
                    wait_idx(base + 16 * (j + 3), r2)
                    compute_t(r2, u_lo)
                    pltpu.async_copy(gsp.at[vbuf.at[r2]], rows.at[r2], gsem)

                r0 = lax.rem(j, NBUF)
                pltpu.make_async_copy(gsp.at[vbuf.at[r0]], rows.at[r0],
                                      gsem).wait()
                pltpu.async_copy(rows.at[r0], acc.at[tbuf.at[r0]], ssem,
                                 add=True)

                @pl.when(j >= 3)
                def _():
                    rm = lax.rem(j - 3, NBUF)
                    pltpu.make_async_copy(rows.at[rm], acc.at[tbuf.at[rm]],
                                          ssem).wait()
                return carry

            lax.fori_loop(0, n_j, body, 0)
            for jj in range(3):
                @pl.when(jj < n_j)
                def _():
                    rt = lax.rem(n_j - 1 - jj, NBUF)
                    pltpu.make_async_copy(rows.at[rt], acc.at[tbuf.at[rt]],
                                          ssem).wait()
            plsc.subcore_barrier()

            pltpu.sync_copy(acc.at[pl.ds(s * (WIN // NS), WIN // NS)],
                            out.at[w, pl.ds(s * (WIN // NS), WIN // NS)])
            plsc.subcore_barrier()


def _norm_body(acc_ref, out_ref):
    p = acc_ref[...].reshape(N_OUT, EMBED)
    nrm = jnp.sqrt(jnp.sum(p * p, axis=1, keepdims=True))
    out_ref[...] = (p / jnp.maximum(nrm, 1e-12))[:N_PREFIX]


def _norm_call(acc9):
    return pl.pallas_call(
        _norm_body,
        out_shape=jax.ShapeDtypeStruct((N_PREFIX, EMBED), jnp.float32),
    )(acc9)


def kernel(x_prefix, x_as, W_prefix, b_prefix, W_as, b_as,
           W_att, b_att, a_att, b_a, edge_u, edge_v):
    del x_prefix, W_prefix, b_prefix, b_att, b_a
    a2 = a_att[EMBED:, :]
    as_out, g_as, b_row = _fmap_call(
        x_as, W_as, b_as.reshape(1, EMBED), W_att, a2, edge_u.reshape(-1, CH))

    acc9 = _edge_kernel(edge_u, edge_v, g_as, b_row)
    prefix_out = _norm_call(acc9)
    return prefix_out, as_out


# speedup vs baseline: 1.1106x; 1.0453x over previous
import functools

import jax
import jax.numpy as jnp
from jax import lax
from jax.experimental import pallas as pl
from jax.experimental.pallas import tpu as pltpu
from jax.experimental.pallas import tpu_sc as plsc

N_PREFIX = 10000
N_AS = 10000
N_EDGES = 320000
EMBED = 64

NC = 2
NS = 16
NW = NC * NS
CH = 128
E_PAD = N_EDGES
NCHUNKS = E_PAD // CH

WIN = 2560
NWIN = 4
ACCR = 2576
TRASH = WIN
NBUF = 8
MAXP = 2

N_OUT = NWIN * WIN
GPAD = 10112


def _fmap_body(x_as_ref, w_as_ref, b_as_ref, w_att_ref, a2_ref, u2d_ref,
               as_out_ref, g_ref, b_ref):
    x = x_as_ref[...]
    h = x @ w_as_ref[...] + b_as_ref[...]
    h = jnp.maximum(h, 0.0)
    nrm = jnp.sqrt(jnp.sum(h * h, axis=1, keepdims=True))
    as_out_ref[...] = h / jnp.maximum(nrm, 1e-12)
    w2 = w_att_ref[...] @ a2_ref[...]
    s = h @ w2
    es = jnp.exp(s - jnp.max(s))
    g_ref[...] = es * h
    u2 = u2d_ref[...]
    lane = lax.broadcasted_iota(jnp.int32, (1, 128), 1)
    row = jnp.where(lane == 8 + (NWIN - 1), NCHUNKS, 0)
    for w in range(1, NWIN):
        cnt = jnp.sum((u2 < w * WIN).astype(jnp.int32))
        row = jnp.where(lane == w, lax.shift_right_logical(cnt, 7), row)
        row = jnp.where(lane == 8 + (w - 1),
                        lax.shift_right_logical(cnt + 127, 7), row)
    b_ref[...] = row


def _fmap_call(x_as, w_as, b_as, w_att, a2, u2d):
    return pl.pallas_call(
        _fmap_body,
        out_shape=(
            jax.ShapeDtypeStruct((N_AS, EMBED), jnp.float32),
            jax.ShapeDtypeStruct((N_AS, EMBED), jnp.float32),
            jax.ShapeDtypeStruct((1, 128), jnp.int32),
        ),
    )(x_as, w_as, b_as, w_att, a2, u2d)


_MESH = plsc.VectorSubcoreMesh(core_axis_name="c", subcore_axis_name="s")


@functools.partial(
    pl.kernel,
    out_type=jax.ShapeDtypeStruct((NWIN, WIN, 128), jnp.float32),
    mesh=_MESH,
    compiler_params=pltpu.CompilerParams(use_tc_tiling_on_sc=False, needs_layout_passes=False),
    scratch_types=[
        pltpu.VMEM((NBUF, CH), jnp.int32),
        pltpu.VMEM((NBUF, CH), jnp.int32),
        pltpu.VMEM((NBUF, CH), jnp.int32),
        pltpu.VMEM((NBUF, CH, EMBED), jnp.float32),
        pltpu.VMEM((ACCR // NS, EMBED), jnp.float32),
        pltpu.VMEM((16,), jnp.int32),
        pltpu.VMEM_SHARED((ACCR, EMBED), jnp.float32),
        pltpu.VMEM_SHARED((N_AS, EMBED), jnp.float32),
        pltpu.SemaphoreType.DMA,
        pltpu.SemaphoreType.DMA,
        pltpu.SemaphoreType.DMA,
    ],
)
def _edge_kernel(u1, v1, g, b2d, out,
                 vbuf, ubuf, tbuf, rows, zbuf, bsm, acc, gsp,
                 isem, gsem, ssem):
    c = lax.axis_index("c")
    s = lax.axis_index("s")
    pltpu.sync_copy(b2d.at[0, pl.ds(0, 16)], bsm)
    gpt = N_AS // NS
    pltpu.sync_copy(g.at[pl.ds(s * gpt, gpt)], gsp.at[pl.ds(s * gpt, gpt)])
    for r in range(ACCR // NS):
        for q in range(EMBED // 16):
            zbuf[r, pl.ds(q * 16, 16)] = jnp.zeros((16,), jnp.float32)

    def start_idx(k, r):
        pltpu.async_copy(u1.at[pl.ds(k * CH, CH)], ubuf.at[r], isem)
        pltpu.async_copy(v1.at[pl.ds(k * CH, CH)], vbuf.at[r], isem)

    def wait_idx(k, r):
        pltpu.make_async_copy(u1.at[pl.ds(k * CH, CH)], ubuf.at[r], isem).wait()
        pltpu.make_async_copy(v1.at[pl.ds(k * CH, CH)], vbuf.at[r], isem).wait()

    def compute_t(r, u_lo):
        for q in range(CH // 16):
            u_vec = ubuf[r, pl.ds(q * 16, 16)]
            d = u_vec - u_lo
            in_win = plsc.bitcast(d, jnp.uint32) < jnp.uint32(WIN)
            tbuf[r, pl.ds(q * 16, 16)] = jnp.where(in_win, d, TRASH)

    for p in range(MAXP):
        w = 2 * p + c

        @pl.when(w < NWIN)
        def _pass():
            u_lo = w * WIN
            bvec = bsm[...]
            lane16 = lax.iota(jnp.int32, 16)
            c_lo = jnp.max(jnp.where(lane16 == w, bvec, 0))
            c_hi = jnp.max(jnp.where(lane16 == 8 + w, bvec, 0))
            nch = c_hi - c_lo
            n_j = lax.shift_right_arithmetic(nch - s + 15, 4)
            base = c_lo + s

            pltpu.sync_copy(zbuf.at[pl.ds(0, ACCR // NS)],
                            acc.at[pl.ds(s * (ACCR // NS), ACCR // NS)])
            plsc.subcore_barrier()

            for jj in range(5):
                @pl.when(jj < n_j)
                def _():
                    start_idx(base + 16 * jj, jj)
            for jj in range(3):
                @pl.when(jj < n_j)
                def _():
                    wait_idx(base + 16 * jj, jj)
                    compute_t(jj, u_lo)
                    pltpu.async_copy(gsp.at[vbuf.at[jj]], rows.at[jj], gsem)

            def body(j, carry):
                @pl.when(j + 5 < n_j)
                def _():
                    start_idx(base + 16 * (j + 5), lax.rem(j + 5, NBUF))

                @pl.when(j + 3 < n_j)
                def _():
                    r2 = lax.rem(j + 3, NBUF)
                    wait_idx(base + 16 * (j + 3), r2)
                    compute_t(r2, u_lo)
                    pltpu.async_copy(gsp.at[vbuf.at[r2]], rows.at[r2], gsem)

                r0 = lax.rem(j, NBUF)
                pltpu.make_async_copy(gsp.at[vbuf.at[r0]], rows.at[r0],
                                      gsem).wait()
                pltpu.async_copy(rows.at[r0], acc.at[tbuf.at[r0]], ssem,
                                 add=True)

                @pl.when(j >= 3)
                def _():
                    rm = lax.rem(j - 3, NBUF)
                    pltpu.make_async_copy(rows.at[rm], acc.at[tbuf.at[rm]],
                                          ssem).wait()
                return carry

            lax.fori_loop(0, n_j, body, 0)
            for jj in range(3):
                @pl.when(jj < n_j)
                def _():
                    rt = lax.rem(n_j - 1 - jj, NBUF)
                    pltpu.make_async_copy(rows.at[rt], acc.at[tbuf.at[rt]],
                                          ssem).wait()
            plsc.subcore_barrier()

            pltpu.sync_copy(
                acc.at[pl.ds(s * (WIN // NS), WIN // NS)],
                out.at[w, pl.ds(s * (WIN // NS), WIN // NS),
                       pl.ds(0, EMBED)])
            plsc.subcore_barrier()


def _norm_body(acc_ref, out_ref):
    p = acc_ref[...].reshape(N_OUT, 128)[:, :EMBED]
    nrm = jnp.sqrt(jnp.sum(p * p, axis=1, keepdims=True))
    out_ref[...] = (p / jnp.maximum(nrm, 1e-12))[:N_PREFIX]


def _norm_call(acc9):
    return pl.pallas_call(
        _norm_body,
        out_shape=jax.ShapeDtypeStruct((N_PREFIX, EMBED), jnp.float32),
    )(acc9)


def kernel(x_prefix, x_as, W_prefix, b_prefix, W_as, b_as,
           W_att, b_att, a_att, b_a, edge_u, edge_v):
    del x_prefix, W_prefix, b_prefix, b_att, b_a
    a2 = a_att[EMBED:, :]
    as_out, g_as, b_row = _fmap_call(
        x_as, W_as, b_as.reshape(1, EMBED), W_att, a2, edge_u.reshape(-1, CH))

    acc9 = _edge_kernel(edge_u, edge_v, g_as, b_row)
    prefix_out = _norm_call(acc9)
    return prefix_out, as_out


# speedup vs baseline: 1.1515x; 1.0368x over previous
import functools

import jax
import jax.numpy as jnp
from jax import lax
from jax.experimental import pallas as pl
from jax.experimental.pallas import tpu as pltpu
from jax.experimental.pallas import tpu_sc as plsc

N_PREFIX = 10000
N_AS = 10000
N_EDGES = 320000
EMBED = 64

NC = 2
NS = 16
NW = NC * NS
CH = 128
E_PAD = N_EDGES
NCHUNKS = E_PAD // CH

WIN = 2560
NWIN = 4
ACCR = 2576
TRASH = WIN
NBUF = 8
MAXP = 2

N_OUT = NWIN * WIN
GPAD = 10112


def _fmap_body(x_as_ref, w_as_ref, b_as_ref, w_att_ref, a2_ref, u2d_ref,
               as_out_ref, g_ref, b_ref):
    x = x_as_ref[...]
    h = x @ w_as_ref[...] + b_as_ref[...]
    h = jnp.maximum(h, 0.0)
    nrm = jnp.sqrt(jnp.sum(h * h, axis=1, keepdims=True))
    as_out_ref[...] = h / jnp.maximum(nrm, 1e-12)
    w2 = w_att_ref[...] @ a2_ref[...]
    s = h @ w2
    es = jnp.exp(s - jnp.max(s))
    g_ref[...] = jnp.concatenate(
        [es * h, jnp.zeros((N_AS, 128 - EMBED), jnp.float32)], axis=1)
    u2 = u2d_ref[...]
    lane = lax.broadcasted_iota(jnp.int32, (1, 128), 1)
    row = jnp.where(lane == 8 + (NWIN - 1), NCHUNKS, 0)
    for w in range(1, NWIN):
        cnt = jnp.sum((u2 < w * WIN).astype(jnp.int32))
        row = jnp.where(lane == w, lax.shift_right_logical(cnt, 7), row)
        row = jnp.where(lane == 8 + (w - 1),
                        lax.shift_right_logical(cnt + 127, 7), row)
    b_ref[...] = row


def _fmap_call(x_as, w_as, b_as, w_att, a2, u2d):
    return pl.pallas_call(
        _fmap_body,
        out_shape=(
            jax.ShapeDtypeStruct((N_AS, EMBED), jnp.float32),
            jax.ShapeDtypeStruct((N_AS, 128), jnp.float32),
            jax.ShapeDtypeStruct((1, 128), jnp.int32),
        ),
    )(x_as, w_as, b_as, w_att, a2, u2d)


_MESH = plsc.VectorSubcoreMesh(core_axis_name="c", subcore_axis_name="s")


@functools.partial(
    pl.kernel,
    out_type=jax.ShapeDtypeStruct((NWIN, WIN, 128), jnp.float32),
    mesh=_MESH,
    compiler_params=pltpu.CompilerParams(use_tc_tiling_on_sc=False, needs_layout_passes=False),
    scratch_types=[
        pltpu.VMEM((NBUF, CH), jnp.int32),
        pltpu.VMEM((NBUF, CH), jnp.int32),
        pltpu.VMEM((NBUF, CH), jnp.int32),
        pltpu.VMEM((NBUF, CH, EMBED), jnp.float32),
        pltpu.VMEM((ACCR // NS, EMBED), jnp.float32),
        pltpu.VMEM((16,), jnp.int32),
        pltpu.VMEM_SHARED((ACCR, EMBED), jnp.float32),
        pltpu.VMEM_SHARED((N_AS, EMBED), jnp.float32),
        pltpu.SemaphoreType.DMA,
        pltpu.SemaphoreType.DMA,
        pltpu.SemaphoreType.DMA,
    ],
)
def _edge_kernel(u1, v1, g, b2d, out,
                 vbuf, ubuf, tbuf, rows, zbuf, bsm, acc, gsp,
                 isem, gsem, ssem):
    c = lax.axis_index("c")
    s = lax.axis_index("s")
    pltpu.sync_copy(b2d.at[0, pl.ds(0, 16)], bsm)
    gpt = N_AS // NS
    pltpu.sync_copy(g.at[pl.ds(s * gpt, gpt), pl.ds(0, EMBED)],
                    gsp.at[pl.ds(s * gpt, gpt)])
    for r in range(ACCR // NS):
        for q in range(EMBED // 16):
            zbuf[r, pl.ds(q * 16, 16)] = jnp.zeros((16,), jnp.float32)

    def start_idx(k, r):
        pltpu.async_copy(u1.at[pl.ds(k * CH, CH)], ubuf.at[r], isem)
        pltpu.async_copy(v1.at[pl.ds(k * CH, CH)], vbuf.at[r], isem)

    def wait_idx(k, r):
        pltpu.make_async_copy(u1.at[pl.ds(k * CH, CH)], ubuf.at[r], isem).wait()
        pltpu.make_async_copy(v1.at[pl.ds(k * CH, CH)], vbuf.at[r], isem).wait()

    def compute_t(r, u_lo):
        for q in range(CH // 16):
            u_vec = ubuf[r, pl.ds(q * 16, 16)]
            d = u_vec - u_lo
            in_win = plsc.bitcast(d, jnp.uint32) < jnp.uint32(WIN)
            tbuf[r, pl.ds(q * 16, 16)] = jnp.where(in_win, d, TRASH)

    for p in range(MAXP):
        w = 2 * p + c

        @pl.when(w < NWIN)
        def _pass():
            u_lo = w * WIN
            bvec = bsm[...]
            lane16 = lax.iota(jnp.int32, 16)
            c_lo = jnp.max(jnp.where(lane16 == w, bvec, 0))
            c_hi = jnp.max(jnp.where(lane16 == 8 + w, bvec, 0))
            nch = c_hi - c_lo
            n_j = lax.shift_right_arithmetic(nch - s + 15, 4)
            base = c_lo + s

            pltpu.sync_copy(zbuf.at[pl.ds(0, ACCR // NS)],
                            acc.at[pl.ds(s * (ACCR // NS), ACCR // NS)])
            plsc.subcore_barrier()

            for jj in range(5):
                @pl.when(jj < n_j)
                def _():
                    start_idx(base + 16 * jj, jj)
            for jj in range(3):
                @pl.when(jj < n_j)
                def _():
                    wait_idx(base + 16 * jj, jj)
                    compute_t(jj, u_lo)
                    pltpu.async_copy(gsp.at[vbuf.at[jj]], rows.at[jj], gsem)

            def body(j, carry):
                @pl.when(j + 5 < n_j)
                def _():
                    start_idx(base + 16 * (j + 5), lax.rem(j + 5, NBUF))

                @pl.when(j + 3 < n_j)
                def _():
                    r2 = lax.rem(j + 3, NBUF)
                    wait_idx(base + 16 * (j + 3), r2)
                    compute_t(r2, u_lo)
                    pltpu.async_copy(gsp.at[vbuf.at[r2]], rows.at[r2], gsem)

                r0 = lax.rem(j, NBUF)
                pltpu.make_async_copy(gsp.at[vbuf.at[r0]], rows.at[r0],
                                      gsem).wait()
                pltpu.async_copy(rows.at[r0], acc.at[tbuf.at[r0]], ssem,
                                 add=True)

                @pl.when(j >= 3)
                def _():
                    rm = lax.rem(j - 3, NBUF)
                    pltpu.make_async_copy(rows.at[rm], acc.at[tbuf.at[rm]],
                                          ssem).wait()
                return carry

            lax.fori_loop(0, n_j, body, 0)
            for jj in range(3):
                @pl.when(jj < n_j)
                def _():
                    rt = lax.rem(n_j - 1 - jj, NBUF)
                    pltpu.make_async_copy(rows.at[rt], acc.at[tbuf.at[rt]],
                                          ssem).wait()
            plsc.subcore_barrier()

            pltpu.sync_copy(
                acc.at[pl.ds(s * (WIN // NS), WIN // NS)],
                out.at[w, pl.ds(s * (WIN // NS), WIN // NS),
                       pl.ds(0, EMBED)])
            plsc.subcore_barrier()


def _norm_body(acc_ref, out_ref):
    p = acc_ref[...].reshape(N_OUT, 128)[:, :EMBED]
    nrm = jnp.sqrt(jnp.sum(p * p, axis=1, keepdims=True))
    out_ref[...] = (p / jnp.maximum(nrm, 1e-12))[:N_PREFIX]


def _norm_call(acc9):
    return pl.pallas_call(
        _norm_body,
        out_shape=jax.ShapeDtypeStruct((N_PREFIX, EMBED), jnp.float32),
    )(acc9)


def kernel(x_prefix, x_as, W_prefix, b_prefix, W_as, b_as,
           W_att, b_att, a_att, b_a, edge_u, edge_v):
    del x_prefix, W_prefix, b_prefix, b_att, b_a
    a2 = a_att[EMBED:, :]
    as_out, g_as, b_row = _fmap_call(
        x_as, W_as, b_as.reshape(1, EMBED), W_att, a2, edge_u.reshape(-1, CH))

    acc9 = _edge_kernel(edge_u, edge_v, g_as, b_row)
    prefix_out = _norm_call(acc9)
    return prefix_out, as_out


# speedup vs baseline: 1.1527x; 1.0011x over previous
import functools

import jax
import jax.numpy as jnp
from jax import lax
from jax.experimental import pallas as pl
from jax.experimental.pallas import tpu as pltpu
from jax.experimental.pallas import tpu_sc as plsc

N_PREFIX = 10000
N_AS = 10000
N_EDGES = 320000
EMBED = 64

NC = 2
NS = 16
CH = 128
E_PAD = N_EDGES
NCHUNKS = E_PAD // CH

WIN = 2560
NWIN = 4
ACCR = 2576
TRASH = WIN
NBUF = 8
MAXP = 2

N_OUT = NWIN * WIN


def _fmap_body(x_as_ref, w_as_ref, b_as_ref, w_att_ref, a2_ref, u2d_ref,
               as_out_ref, g_ref, b_ref):
    x = x_as_ref[...]
    h = x @ w_as_ref[...] + b_as_ref[...]
    h = jnp.maximum(h, 0.0)
    nrm = jnp.sqrt(jnp.sum(h * h, axis=1, keepdims=True))
    as_out_ref[...] = h / jnp.maximum(nrm, 1e-12)
    w2 = w_att_ref[...] @ a2_ref[...]
    s = h @ w2
    es = jnp.exp(s - jnp.max(s))
    g_ref[...] = jnp.concatenate(
        [es * h, jnp.zeros((N_AS, 128 - EMBED), jnp.float32)], axis=1)
    u2 = u2d_ref[...]
    lane = lax.broadcasted_iota(jnp.int32, (1, 128), 1)
    row = jnp.where(lane == 8 + (NWIN - 1), NCHUNKS, 0)
    for w in range(1, NWIN):
        cnt = jnp.sum((u2 < w * WIN).astype(jnp.int32))
        row = jnp.where(lane == w, lax.shift_right_logical(cnt, 7), row)
        row = jnp.where(lane == 8 + (w - 1),
                        lax.shift_right_logical(cnt + 127, 7), row)
    b_ref[...] = row


def _fmap_call(x_as, w_as, b_as, w_att, a2, u2d):
    return pl.pallas_call(
        _fmap_body,
        out_shape=(
            jax.ShapeDtypeStruct((N_AS, EMBED), jnp.float32),
            jax.ShapeDtypeStruct((N_AS, 128), jnp.float32),
            jax.ShapeDtypeStruct((1, 128), jnp.int32),
        ),
    )(x_as, w_as, b_as, w_att, a2, u2d)


_MESH = plsc.VectorSubcoreMesh(core_axis_name="c", subcore_axis_name="s")


@functools.partial(
    pl.kernel,
    out_type=jax.ShapeDtypeStruct((NWIN, WIN, 128), jnp.float32),
    mesh=_MESH,
    compiler_params=pltpu.CompilerParams(use_tc_tiling_on_sc=False, needs_layout_passes=False),
    scratch_types=[
        pltpu.VMEM((NBUF, CH), jnp.int32),
        pltpu.VMEM((NBUF, CH), jnp.int32),
        pltpu.VMEM((NBUF, CH), jnp.int32),
        pltpu.VMEM((NBUF, CH, EMBED), jnp.float32),
        pltpu.VMEM((ACCR // NS, EMBED), jnp.float32),
        pltpu.VMEM((16,), jnp.int32),
        pltpu.VMEM_SHARED((ACCR, EMBED), jnp.float32),
        pltpu.VMEM_SHARED((N_AS, EMBED), jnp.float32),
        pltpu.SemaphoreType.DMA,
        pltpu.SemaphoreType.DMA,
        pltpu.SemaphoreType.DMA,
    ],
)
def _edge_kernel(u1, v1, g, b2d, out,
                 vbuf, ubuf, tbuf, rows, zbuf, bsm, acc, gsp,
                 isem, gsem, ssem):
    c = lax.axis_index("c")
    s = lax.axis_index("s")
    pltpu.sync_copy(b2d.at[0, pl.ds(0, 16)], bsm)
    gpt = N_AS // NS
    pltpu.sync_copy(g.at[pl.ds(s * gpt, gpt), pl.ds(0, EMBED)],
                    gsp.at[pl.ds(s * gpt, gpt)])
    for r in range(ACCR // NS):
        for q in range(EMBED // 16):
            zbuf[r, pl.ds(q * 16, 16)] = jnp.zeros((16,), jnp.float32)

    def start_idx(k, r):
        pltpu.async_copy(u1.at[pl.ds(k * CH, CH)], ubuf.at[r], isem)
        pltpu.async_copy(v1.at[pl.ds(k * CH, CH)], vbuf.at[r], isem)

    def wait_idx(k, r):
        pltpu.make_async_copy(u1.at[pl.ds(k * CH, CH)], ubuf.at[r], isem).wait()
        pltpu.make_async_copy(v1.at[pl.ds(k * CH, CH)], vbuf.at[r], isem).wait()

    def compute_t(r, u_lo):
        for q in range(CH // 16):
            u_vec = ubuf[r, pl.ds(q * 16, 16)]
            d = u_vec - u_lo
            in_win = plsc.bitcast(d, jnp.uint32) < jnp.uint32(WIN)
            tbuf[r, pl.ds(q * 16, 16)] = jnp.where(in_win, d, TRASH)

    for p in range(MAXP):
        w = 2 * p + c

        @pl.when(w < NWIN)
        def _pass():
            u_lo = w * WIN
            bvec = bsm[...]
            lane16 = lax.iota(jnp.int32, 16)
            c_lo = jnp.max(jnp.where(lane16 == w, bvec, 0))
            c_hi = jnp.max(jnp.where(lane16 == 8 + w, bvec, 0))
            nch = c_hi - c_lo
            n_j = lax.shift_right_arithmetic(nch - s + 15, 4)
            base = c_lo + s

            pltpu.sync_copy(zbuf.at[pl.ds(0, ACCR // NS)],
                            acc.at[pl.ds(s * (ACCR // NS), ACCR // NS)])
            plsc.subcore_barrier()

            for jj in range(5):
                @pl.when(jj < n_j)
                def _():
                    start_idx(base + 16 * jj, jj)
            for jj in range(3):
                @pl.when(jj < n_j)
                def _():
                    wait_idx(base + 16 * jj, jj)
                    compute_t(jj, u_lo)
                    pltpu.async_copy(gsp.at[vbuf.at[jj]], rows.at[jj], gsem)

            def body(j, carry):
                @pl.when(j + 5 < n_j)
                def _():
                    start_idx(base + 16 * (j + 5), lax.rem(j + 5, NBUF))

                @pl.when(j + 3 < n_j)
                def _():
                    r2 = lax.rem(j + 3, NBUF)
                    wait_idx(base + 16 * (j + 3), r2)
                    compute_t(r2, u_lo)
                    pltpu.async_copy(gsp.at[vbuf.at[r2]], rows.at[r2], gsem)

                r0 = lax.rem(j, NBUF)
                pltpu.make_async_copy(gsp.at[vbuf.at[r0]], rows.at[r0],
                                      gsem).wait()
                pltpu.async_copy(rows.at[r0], acc.at[tbuf.at[r0]], ssem,
                                 add=True)

                @pl.when(j >= 3)
                def _():
                    rm = lax.rem(j - 3, NBUF)
                    pltpu.make_async_copy(rows.at[rm], acc.at[tbuf.at[rm]],
                                          ssem).wait()
                return carry

            lax.fori_loop(0, n_j, body, 0)
            for jj in range(3):
                @pl.when(jj < n_j)
                def _():
                    rt = lax.rem(n_j - 1 - jj, NBUF)
                    pltpu.make_async_copy(rows.at[rt], acc.at[tbuf.at[rt]],
                                          ssem).wait()
            plsc.subcore_barrier()

            pltpu.sync_copy(
                acc.at[pl.ds(s * (WIN // NS), WIN // NS)],
                out.at[w, pl.ds(s * (WIN // NS), WIN // NS),
                       pl.ds(0, EMBED)])
            plsc.subcore_barrier()


def _norm_body(acc_ref, out_ref):
    p = acc_ref[...].reshape(N_OUT, 128)[:, :EMBED]
    nrm = jnp.sqrt(jnp.sum(p * p, axis=1, keepdims=True))
    out_ref[...] = (p / jnp.maximum(nrm, 1e-12))[:N_PREFIX]


def _norm_call(acc9):
    return pl.pallas_call(
        _norm_body,
        out_shape=jax.ShapeDtypeStruct((N_PREFIX, EMBED), jnp.float32),
    )(acc9)


def kernel(x_prefix, x_as, W_prefix, b_prefix, W_as, b_as,
           W_att, b_att, a_att, b_a, edge_u, edge_v):
    del x_prefix, W_prefix, b_prefix, b_att, b_a
    a2 = a_att[EMBED:, :]
    as_out, g_as, b_row = _fmap_call(
        x_as, W_as, b_as.reshape(1, EMBED), W_att, a2, edge_u.reshape(-1, CH))

    acc9 = _edge_kernel(edge_u, edge_v, g_as, b_row)
    prefix_out = _norm_call(acc9)
    return prefix_out, as_out
